# Initial kernel scaffold; baseline (speedup 1.0000x reference)
import jax
import jax.numpy as jnp
from jax.experimental import pallas as pl


def kernel(pc, params):
    raise NotImplementedError("write your pallas kernel here")


# speedup vs baseline: 6.8888x; 6.8888x over previous
import jax
import jax.numpy as jnp
from jax.experimental import pallas as pl
from jax.experimental.pallas import tpu as pltpu

_LAYER_CFGS = [
    (8, 1, -1, 0, 48, 24, 4, False),
    (12, 2, 384, 48, 96, 12, 2, False),
    (16, 2, 128, 96, 192, 24, 2, False),
    (16, 3, -1, 192, 384, 48, 2, True),
]


def _elu(x):
    return jnp.where(x > 0, x, jnp.exp(x) - 1.0)


def _make_layer_kernel(K, D, dm, Cin, with_global, N, P_tile):
    KD = K * D

    def body(rep_ref, ptsT_ref, cat_ref, Wd1_ref, bd1_ref, Wd2_ref, bd2_ref,
             Wx0_ref, bx0_ref, Wx1_ref, bx1_ref, Wx2_ref, bx2_ref,
             Wdw_ref, Wpw_ref, bpw_ref, *rest):
        if with_global:
            Wg1_ref, bg1_ref, Wg2_ref, bg2_ref, out_ref = rest
        else:
            (out_ref,) = rest

        rep = rep_ref[0]
        cat = cat_ref[0]

        dx = rep[:, 0:1] - ptsT_ref[0, 0:1, :]
        dy = rep[:, 1:2] - ptsT_ref[0, 1:2, :]
        dz = rep[:, 2:3] - ptsT_ref[0, 2:3, :]
        d2 = (dx * dx + dy * dy) + dz * dz

        iota = jax.lax.broadcasted_iota(jnp.int32, (P_tile, N), 1)
        inf = jnp.float32(jnp.inf)
        nn = []
        for t in range(KD):
            m = jnp.min(d2, axis=1, keepdims=True)
            cand = jnp.where(d2 == m, iota, N)
            sel = jnp.min(cand, axis=1, keepdims=True)
            onehot = iota == sel
            if t % D == 0:
                nn.append(jnp.dot(onehot.astype(jnp.float32), cat,
                                  preferred_element_type=jnp.float32))
            d2 = jnp.where(onehot, inf, d2)

        local = [g[:, 0:3] - rep for g in nn]
        lf = jnp.concatenate(local, axis=1)
        loc_all = jnp.concatenate(local, axis=0)

        Wd1 = Wd1_ref[...]
        h = (loc_all[:, 0:1] * Wd1[0:1, :] + loc_all[:, 1:2] * Wd1[1:2, :]
             + loc_all[:, 2:3] * Wd1[2:3, :]) + bd1_ref[...]
        h = _elu(h)
        h = _elu(jnp.dot(h, Wd2_ref[...], preferred_element_type=jnp.float32)
                 + bd2_ref[...])

        X = _elu(jnp.dot(lf, Wx0_ref[...], preferred_element_type=jnp.float32)
                 + bx0_ref[...])
        X = _elu(jnp.dot(X, Wx1_ref[...], preferred_element_type=jnp.float32)
                 + bx1_ref[...])
        X = jnp.dot(X, Wx2_ref[...], preferred_element_type=jnp.float32) \
            + bx2_ref[...]

        H = []
        for j in range(K):
            hj = h[j * P_tile:(j + 1) * P_tile]
            if Cin:
                hj = jnp.concatenate([hj, nn[j][:, 3:]], axis=1)
            H.append(hj)

        fX = []
        for k in range(K):
            acc = X[:, k * K:k * K + 1] * H[0]
            for j in range(1, K):
                acc = acc + X[:, k * K + j:k * K + j + 1] * H[j]
            fX.append(acc)

        Wdw = Wdw_ref[...]
        dws = []
        for mi in range(dm):
            w = Wdw[mi]
            acc = fX[0] * w[0:1, :]
            for k in range(1, K):
                acc = acc + fX[k] * w[k:k + 1, :]
            dws.append(acc)
        dw = jnp.concatenate(dws, axis=1)
        out = jnp.dot(dw, Wpw_ref[...], preferred_element_type=jnp.float32) \
            + bpw_ref[...]

        if with_global:
            Wg1 = Wg1_ref[...]
            g = (rep[:, 0:1] * Wg1[0:1, :] + rep[:, 1:2] * Wg1[1:2, :]
                 + rep[:, 2:3] * Wg1[2:3, :]) + bg1_ref[...]
            g = _elu(g)
            g = _elu(jnp.dot(g, Wg2_ref[...],
                             preferred_element_type=jnp.float32)
                     + bg2_ref[...])
            out = jnp.concatenate([g, out], axis=1)

        out_ref[0] = out

    return body


def _xconv_layer(pts, fts, rep, p, K, D, dm, with_global):
    B, N, _ = pts.shape
    P = rep.shape[1]
    Cin = 0 if fts is None else fts.shape[-1]
    Cd = p['Wd1'].shape[1]
    Cmid = Cd + Cin
    Cout = p['Wpw'].shape[1]
    Cg = p['Wg1'].shape[1] if with_global else 0

    P_tile = 256 if P > 384 else P

    ptsT = jnp.transpose(pts, (0, 2, 1))
    cat = pts if fts is None else jnp.concatenate([pts, fts], axis=-1)
    Wdw_t = jnp.transpose(p['Wdw'], (2, 0, 1))
    Wpw_perm = p['Wpw'].reshape(Cmid, dm, Cout).transpose(1, 0, 2) \
        .reshape(dm * Cmid, Cout)

    def b2(b):
        return b.reshape(1, -1)

    ops = [rep, ptsT, cat,
           p['Wd1'], b2(p['bd1']), p['Wd2'], b2(p['bd2']),
           p['Wx0'], b2(p['bx0']), p['Wx1'], b2(p['bx1']),
           p['Wx2'], b2(p['bx2']),
           Wdw_t, Wpw_perm, b2(p['bpw'])]
    if with_global:
        ops += [p['Wg1'], b2(p['bg1']), p['Wg2'], b2(p['bg2'])]

    def wspec(arr):
        zeros = (0,) * arr.ndim
        return pl.BlockSpec(arr.shape, lambda b, j, _z=zeros: _z)

    in_specs = [
        pl.BlockSpec((1, P_tile, 3), lambda b, j: (b, j, 0)),
        pl.BlockSpec((1, 3, N), lambda b, j: (b, 0, 0)),
        pl.BlockSpec((1, N, 3 + Cin), lambda b, j: (b, 0, 0)),
    ] + [wspec(a) for a in ops[3:]]

    fn = _make_layer_kernel(K, D, dm, Cin, with_global, N, P_tile)
    return pl.pallas_call(
        fn,
        grid=(B, P // P_tile),
        in_specs=in_specs,
        out_specs=pl.BlockSpec((1, P_tile, Cg + Cout), lambda b, j: (b, j, 0)),
        out_shape=jax.ShapeDtypeStruct((B, P, Cg + Cout), jnp.float32),
        compiler_params=pltpu.CompilerParams(
            dimension_semantics=("parallel", "parallel")),
    )(*ops)


def kernel(pc, params):
    pts = pc
    fts = None
    for (K, D, P, _Cin, _Cout, _Cd, dm, wg), p in zip(_LAYER_CFGS, params):
        rep = pts if P == -1 else pts[:, :P, :]
        fts = _xconv_layer(pts, fts, rep, p, K, D, dm, wg)
        pts = rep
    return fts


# speedup vs baseline: 14.4592x; 2.0989x over previous
import jax
import jax.numpy as jnp
from jax.experimental import pallas as pl
from jax.experimental.pallas import tpu as pltpu

_LAYER_CFGS = [
    (8, 1, -1, 0, 48, 24, 4, False),
    (12, 2, 384, 48, 96, 12, 2, False),
    (16, 2, 128, 96, 192, 24, 2, False),
    (16, 3, -1, 192, 384, 48, 2, True),
]


def _elu(x):
    return jnp.where(x > 0, x, jnp.exp(x) - 1.0)


def _make_layer_kernel(K, D, dm, Cin, with_global, N, P_tile):
    KD = K * D

    def body(pts_ref, repT_ref, catT_ref, Wd1_ref, bd1_ref, Wd2_ref, bd2_ref,
             Wx0_ref, bx0_ref, Wx1_ref, bx1_ref, Wx2_ref, bx2_ref,
             Wdw_ref, Wpw_ref, bpw_ref, *rest):
        if with_global:
            Wg1_ref, bg1_ref, Wg2_ref, bg2_ref, out_ref = rest
        else:
            (out_ref,) = rest

        pts = pts_ref[0]
        repT = repT_ref[0]
        catT = catT_ref[0]

        dx = pts[:, 0:1] - repT[0:1, :]
        dy = pts[:, 1:2] - repT[1:2, :]
        dz = pts[:, 2:3] - repT[2:3, :]
        d2 = (dx * dx + dy * dy) + dz * dz

        iota = jax.lax.broadcasted_iota(jnp.int32, (N, P_tile), 0)
        inf = jnp.float32(jnp.inf)
        nnT = []
        for t in range(KD):
            m = jnp.min(d2, axis=0, keepdims=True)
            cand = jnp.where(d2 == m, iota, N)
            sel = jnp.min(cand, axis=0, keepdims=True)
            onehot = iota == sel
            if t % D == 0:
                nnT.append(jnp.dot(catT, onehot.astype(jnp.float32),
                                   preferred_element_type=jnp.float32))
            d2 = jnp.where(onehot, inf, d2)

        locT = [g[0:3, :] - repT for g in nnT]
        lfT = jnp.concatenate(locT, axis=0)
        locT_all = jnp.concatenate(locT, axis=1)

        Wd1 = Wd1_ref[...]
        h = (Wd1[:, 0:1] * locT_all[0:1, :] + Wd1[:, 1:2] * locT_all[1:2, :]
             + Wd1[:, 2:3] * locT_all[2:3, :]) + bd1_ref[...]
        h = _elu(h)
        h = _elu(jnp.dot(Wd2_ref[...], h, preferred_element_type=jnp.float32)
                 + bd2_ref[...])

        X = _elu(jnp.dot(Wx0_ref[...], lfT, preferred_element_type=jnp.float32)
                 + bx0_ref[...])
        X = _elu(jnp.dot(Wx1_ref[...], X, preferred_element_type=jnp.float32)
                 + bx1_ref[...])
        X = jnp.dot(Wx2_ref[...], X, preferred_element_type=jnp.float32) \
            + bx2_ref[...]

        H = []
        for j in range(K):
            hj = h[:, j * P_tile:(j + 1) * P_tile]
            if Cin:
                hj = jnp.concatenate([hj, nnT[j][3:, :]], axis=0)
            H.append(hj)

        fX = []
        for k in range(K):
            acc = X[k * K:k * K + 1, :] * H[0]
            for j in range(1, K):
                acc = acc + X[k * K + j:k * K + j + 1, :] * H[j]
            fX.append(acc)

        Wdw = Wdw_ref[...]
        dws = []
        for mi in range(dm):
            w = Wdw[mi]
            acc = fX[0] * w[:, 0:1]
            for k in range(1, K):
                acc = acc + fX[k] * w[:, k:k + 1]
            dws.append(acc)
        dwT = jnp.concatenate(dws, axis=0)
        out = jnp.dot(Wpw_ref[...], dwT, preferred_element_type=jnp.float32) \
            + bpw_ref[...]

        if with_global:
            Wg1 = Wg1_ref[...]
            g = (Wg1[:, 0:1] * repT[0:1, :] + Wg1[:, 1:2] * repT[1:2, :]
                 + Wg1[:, 2:3] * repT[2:3, :]) + bg1_ref[...]
            g = _elu(g)
            g = _elu(jnp.dot(Wg2_ref[...], g,
                             preferred_element_type=jnp.float32)
                     + bg2_ref[...])
            out = jnp.concatenate([g, out], axis=0)

        out_ref[0] = out

    return body


def _xconv_layer(pts, ptsT, ftsT, p, K, D, P, dm, with_global):
    B, N, _ = pts.shape
    Cin = 0 if ftsT is None else ftsT.shape[1]
    Cd = p['Wd1'].shape[1]
    Cmid = Cd + Cin
    Cout = p['Wpw'].shape[1]
    Cg = p['Wg1'].shape[1] if with_global else 0

    P_tile = 256 if P > 384 else P

    catT = ptsT if ftsT is None else jnp.concatenate([ptsT, ftsT], axis=1)
    Wdw_p = jnp.transpose(p['Wdw'], (2, 1, 0))
    WpwT = p['Wpw'].reshape(Cmid, dm, Cout).transpose(1, 0, 2) \
        .reshape(dm * Cmid, Cout).T

    def bcol(b):
        return b.reshape(-1, 1)

    ops = [pts, ptsT, catT,
           p['Wd1'].T, bcol(p['bd1']), p['Wd2'].T, bcol(p['bd2']),
           p['Wx0'].T, bcol(p['bx0']), p['Wx1'].T, bcol(p['bx1']),
           p['Wx2'].T, bcol(p['bx2']),
           Wdw_p, WpwT, bcol(p['bpw'])]
    if with_global:
        ops += [p['Wg1'].T, bcol(p['bg1']), p['Wg2'].T, bcol(p['bg2'])]

    def wspec(arr):
        zeros = (0,) * arr.ndim
        return pl.BlockSpec(arr.shape, lambda b, j, _z=zeros: _z)

    in_specs = [
        pl.BlockSpec((1, N, 3), lambda b, j: (b, 0, 0)),
        pl.BlockSpec((1, 3, P_tile), lambda b, j: (b, 0, j)),
        pl.BlockSpec((1, 3 + Cin, N), lambda b, j: (b, 0, 0)),
    ] + [wspec(a) for a in ops[3:]]

    fn = _make_layer_kernel(K, D, dm, Cin, with_global, N, P_tile)
    return pl.pallas_call(
        fn,
        grid=(B, P // P_tile),
        in_specs=in_specs,
        out_specs=pl.BlockSpec((1, Cg + Cout, P_tile), lambda b, j: (b, 0, j)),
        out_shape=jax.ShapeDtypeStruct((B, Cg + Cout, P), jnp.float32),
        compiler_params=pltpu.CompilerParams(
            dimension_semantics=("parallel", "parallel")),
    )(*ops)


def kernel(pc, params):
    pcT = jnp.transpose(pc, (0, 2, 1))
    N = pc.shape[1]
    ftsT = None
    for (K, D, P, _Cin, _Cout, _Cd, dm, wg), p in zip(_LAYER_CFGS, params):
        P_l = N if P == -1 else P
        ftsT = _xconv_layer(pc[:, :N, :], pcT[:, :, :N], ftsT, p,
                            K, D, P_l, dm, wg)
        N = P_l
    return jnp.transpose(ftsT, (0, 2, 1))


# speedup vs baseline: 17.2460x; 1.1927x over previous
import jax
import jax.numpy as jnp
from jax.experimental import pallas as pl
from jax.experimental.pallas import tpu as pltpu

_LAYER_CFGS = [
    (8, 1, -1, 0, 48, 24, 4, False),
    (12, 2, 384, 48, 96, 12, 2, False),
    (16, 2, 128, 96, 192, 24, 2, False),
    (16, 3, -1, 192, 384, 48, 2, True),
]


def _elu(x):
    return jnp.where(x > 0, x, jnp.exp(x) - 1.0)


def _make_layer_kernel(K, D, dm, Cin, with_global, N, P_tile):
    KD = K * D

    def body(pts_ref, repT_ref, catT_ref, Wd1_ref, bd1_ref, Wd2_ref, bd2_ref,
             Wx0_ref, bx0_ref, Wx1_ref, bx1_ref, Wx2_ref, bx2_ref,
             Wdw_ref, Wpw_ref, bpw_ref, *rest):
        if with_global:
            Wg1_ref, bg1_ref, Wg2_ref, bg2_ref, out_ref = rest
        else:
            (out_ref,) = rest

        pts = pts_ref[0]
        repT = repT_ref[0]
        catT = catT_ref[0]
        Ccat = catT.shape[0]

        ones_row = jnp.ones((1, P_tile), jnp.float32)
        dx = jnp.dot(pts[:, 0:1], ones_row,
                     preferred_element_type=jnp.float32) - repT[0:1, :]
        dy = jnp.dot(pts[:, 1:2], ones_row,
                     preferred_element_type=jnp.float32) - repT[1:2, :]
        dz = jnp.dot(pts[:, 2:3], ones_row,
                     preferred_element_type=jnp.float32) - repT[2:3, :]
        d2 = (dx * dx + dy * dy) + dz * dz

        iota = jax.lax.broadcasted_iota(jnp.int32, (N, P_tile), 0)
        inf = jnp.float32(jnp.inf)
        ones_cnt = jnp.ones((1, N), jnp.float32)
        d2m = d2
        nn_fast = []
        maxcnt = jnp.float32(0.0)
        for t in range(KD):
            m = jnp.min(d2m, axis=0, keepdims=True)
            eq = d2m == m
            eqf = eq.astype(jnp.float32)
            if t % D == 0:
                gat = jnp.dot(catT, eqf, preferred_element_type=jnp.float32)
                nn_fast.append(gat)
            cnt = jnp.dot(ones_cnt, eqf, preferred_element_type=jnp.float32)
            maxcnt = jnp.maximum(maxcnt, jnp.max(cnt))
            d2m = jnp.where(eq, inf, d2m)
        fast = jnp.concatenate(nn_fast, axis=0)

        def _exact(_):
            d2e = d2
            outs = []
            for t in range(KD):
                m = jnp.min(d2e, axis=0, keepdims=True)
                cand = jnp.where(d2e == m, iota, N)
                sel = jnp.min(cand, axis=0, keepdims=True)
                onehot = iota == sel
                if t % D == 0:
                    outs.append(jnp.dot(catT, onehot.astype(jnp.float32),
                                        preferred_element_type=jnp.float32))
                d2e = jnp.where(onehot, inf, d2e)
            return jnp.concatenate(outs, axis=0)

        gathered = jax.lax.cond(maxcnt > 1.5, _exact, lambda _: fast, None)
        nnT = [gathered[k * Ccat:(k + 1) * Ccat] for k in range(K)]

        locT = [g[0:3, :] - repT for g in nnT]
        lfT = jnp.concatenate(locT, axis=0)
        locT_all = jnp.concatenate(locT, axis=1)

        Wd1 = Wd1_ref[...]
        h = (Wd1[:, 0:1] * locT_all[0:1, :] + Wd1[:, 1:2] * locT_all[1:2, :]
             + Wd1[:, 2:3] * locT_all[2:3, :]) + bd1_ref[...]
        h = _elu(h)
        h = _elu(jnp.dot(Wd2_ref[...], h, preferred_element_type=jnp.float32)
                 + bd2_ref[...])

        X = _elu(jnp.dot(Wx0_ref[...], lfT, preferred_element_type=jnp.float32)
                 + bx0_ref[...])
        X = _elu(jnp.dot(Wx1_ref[...], X, preferred_element_type=jnp.float32)
                 + bx1_ref[...])
        X = jnp.dot(Wx2_ref[...], X, preferred_element_type=jnp.float32) \
            + bx2_ref[...]

        H = []
        for j in range(K):
            hj = h[:, j * P_tile:(j + 1) * P_tile]
            if Cin:
                hj = jnp.concatenate([hj, nnT[j][3:, :]], axis=0)
            H.append(hj)

        fX = []
        for k in range(K):
            acc = X[k * K:k * K + 1, :] * H[0]
            for j in range(1, K):
                acc = acc + X[k * K + j:k * K + j + 1, :] * H[j]
            fX.append(acc)

        Wdw = Wdw_ref[...]
        dws = []
        for mi in range(dm):
            w = Wdw[mi]
            acc = fX[0] * w[:, 0:1]
            for k in range(1, K):
                acc = acc + fX[k] * w[:, k:k + 1]
            dws.append(acc)
        dwT = jnp.concatenate(dws, axis=0)
        out = jnp.dot(Wpw_ref[...], dwT, preferred_element_type=jnp.float32) \
            + bpw_ref[...]

        if with_global:
            Wg1 = Wg1_ref[...]
            g = (Wg1[:, 0:1] * repT[0:1, :] + Wg1[:, 1:2] * repT[1:2, :]
                 + Wg1[:, 2:3] * repT[2:3, :]) + bg1_ref[...]
            g = _elu(g)
            g = _elu(jnp.dot(Wg2_ref[...], g,
                             preferred_element_type=jnp.float32)
                     + bg2_ref[...])
            out = jnp.concatenate([g, out], axis=0)

        out_ref[0] = out

    return body


def _xconv_layer(pts, ptsT, ftsT, p, K, D, P, dm, with_global):
    B, N, _ = pts.shape
    Cin = 0 if ftsT is None else ftsT.shape[1]
    Cd = p['Wd1'].shape[1]
    Cmid = Cd + Cin
    Cout = p['Wpw'].shape[1]
    Cg = p['Wg1'].shape[1] if with_global else 0

    P_tile = 512 if P > 384 else P

    catT = ptsT if ftsT is None else jnp.concatenate([ptsT, ftsT], axis=1)
    Wdw_p = jnp.transpose(p['Wdw'], (2, 1, 0))
    WpwT = p['Wpw'].reshape(Cmid, dm, Cout).transpose(1, 0, 2) \
        .reshape(dm * Cmid, Cout).T

    def bcol(b):
        return b.reshape(-1, 1)

    ops = [pts, ptsT, catT,
           p['Wd1'].T, bcol(p['bd1']), p['Wd2'].T, bcol(p['bd2']),
           p['Wx0'].T, bcol(p['bx0']), p['Wx1'].T, bcol(p['bx1']),
           p['Wx2'].T, bcol(p['bx2']),
           Wdw_p, WpwT, bcol(p['bpw'])]
    if with_global:
        ops += [p['Wg1'].T, bcol(p['bg1']), p['Wg2'].T, bcol(p['bg2'])]

    def wspec(arr):
        zeros = (0,) * arr.ndim
        return pl.BlockSpec(arr.shape, lambda b, j, _z=zeros: _z)

    in_specs = [
        pl.BlockSpec((1, N, 3), lambda b, j: (b, 0, 0)),
        pl.BlockSpec((1, 3, P_tile), lambda b, j: (b, 0, j)),
        pl.BlockSpec((1, 3 + Cin, N), lambda b, j: (b, 0, 0)),
    ] + [wspec(a) for a in ops[3:]]

    fn = _make_layer_kernel(K, D, dm, Cin, with_global, N, P_tile)
    return pl.pallas_call(
        fn,
        grid=(B, P // P_tile),
        in_specs=in_specs,
        out_specs=pl.BlockSpec((1, Cg + Cout, P_tile), lambda b, j: (b, 0, j)),
        out_shape=jax.ShapeDtypeStruct((B, Cg + Cout, P), jnp.float32),
        compiler_params=pltpu.CompilerParams(
            dimension_semantics=("parallel", "parallel")),
    )(*ops)


def kernel(pc, params):
    pcT = jnp.transpose(pc, (0, 2, 1))
    N = pc.shape[1]
    ftsT = None
    for (K, D, P, _Cin, _Cout, _Cd, dm, wg), p in zip(_LAYER_CFGS, params):
        P_l = N if P == -1 else P
        ftsT = _xconv_layer(pc[:, :N, :], pcT[:, :, :N], ftsT, p,
                            K, D, P_l, dm, wg)
        N = P_l
    return jnp.transpose(ftsT, (0, 2, 1))


# speedup vs baseline: 17.6748x; 1.0249x over previous
import jax
import jax.numpy as jnp
from jax.experimental import pallas as pl
from jax.experimental.pallas import tpu as pltpu

_LAYER_CFGS = [
    (8, 1, -1, 0, 48, 24, 4, False),
    (12, 2, 384, 48, 96, 12, 2, False),
    (16, 2, 128, 96, 192, 24, 2, False),
    (16, 3, -1, 192, 384, 48, 2, True),
]


def _elu(x):
    return jnp.where(x > 0, x, jnp.exp(x) - 1.0)


def _make_layer_kernel(K, D, dm, Cin, with_global, N, P_tile):
    KD = K * D

    def body(pts_ref, repT_ref, catT_ref, Wd1_ref, bd1_ref, Wd2_ref, bd2_ref,
             Wx0_ref, bx0_ref, Wx1_ref, bx1_ref, Wx2_ref, bx2_ref,
             Wdw_ref, Wpw_ref, bpw_ref, *rest):
        if with_global:
            Wg1_ref, bg1_ref, Wg2_ref, bg2_ref, out_ref = rest
        else:
            (out_ref,) = rest

        pts = pts_ref[0]
        repT = repT_ref[0]
        catT = catT_ref[0]
        Ccat = catT.shape[0]

        dx = pts[:, 0:1] - repT[0:1, :]
        dy = pts[:, 1:2] - repT[1:2, :]
        dz = pts[:, 2:3] - repT[2:3, :]
        d2 = (dx * dx + dy * dy) + dz * dz

        iota = jax.lax.broadcasted_iota(jnp.int32, (N, P_tile), 0)
        inf = jnp.float32(jnp.inf)
        ones_cnt = jnp.ones((1, N), jnp.float32)
        d2m = d2
        nn_fast = []
        maxcnt = jnp.float32(0.0)
        for t in range(KD):
            m = jnp.min(d2m, axis=0, keepdims=True)
            eq = d2m == m
            eqf = eq.astype(jnp.float32)
            if t % D == 0:
                gat = jnp.dot(catT, eqf, preferred_element_type=jnp.float32)
                nn_fast.append(gat)
            cnt = jnp.dot(ones_cnt, eqf, preferred_element_type=jnp.float32)
            maxcnt = jnp.maximum(maxcnt, jnp.max(cnt))
            d2m = jnp.where(eq, inf, d2m)
        fast = jnp.concatenate(nn_fast, axis=0)

        def _exact(_):
            d2e = d2
            outs = []
            for t in range(KD):
                m = jnp.min(d2e, axis=0, keepdims=True)
                cand = jnp.where(d2e == m, iota, N)
                sel = jnp.min(cand, axis=0, keepdims=True)
                onehot = iota == sel
                if t % D == 0:
                    outs.append(jnp.dot(catT, onehot.astype(jnp.float32),
                                        preferred_element_type=jnp.float32))
                d2e = jnp.where(onehot, inf, d2e)
            return jnp.concatenate(outs, axis=0)

        gathered = jax.lax.cond(maxcnt > 1.5, _exact, lambda _: fast, None)
        nnT = [gathered[k * Ccat:(k + 1) * Ccat] for k in range(K)]

        locT = [g[0:3, :] - repT for g in nnT]
        lfT = jnp.concatenate(locT, axis=0)
        locT_all = jnp.concatenate(locT, axis=1)

        Wd1 = Wd1_ref[...]
        h = (Wd1[:, 0:1] * locT_all[0:1, :] + Wd1[:, 1:2] * locT_all[1:2, :]
             + Wd1[:, 2:3] * locT_all[2:3, :]) + bd1_ref[...]
        h = _elu(h)
        h = _elu(jnp.dot(Wd2_ref[...], h, preferred_element_type=jnp.float32)
                 + bd2_ref[...])

        X = _elu(jnp.dot(Wx0_ref[...], lfT, preferred_element_type=jnp.float32)
                 + bx0_ref[...])
        X = _elu(jnp.dot(Wx1_ref[...], X, preferred_element_type=jnp.float32)
                 + bx1_ref[...])
        X = jnp.dot(Wx2_ref[...], X, preferred_element_type=jnp.float32) \
            + bx2_ref[...]

        H = []
        for j in range(K):
            hj = h[:, j * P_tile:(j + 1) * P_tile]
            if Cin:
                hj = jnp.concatenate([hj, nnT[j][3:, :]], axis=0)
            H.append(hj)

        fX = []
        for k in range(K):
            acc = X[k * K:k * K + 1, :] * H[0]
            for j in range(1, K):
                acc = acc + X[k * K + j:k * K + j + 1, :] * H[j]
            fX.append(acc)

        Wdw = Wdw_ref[...]
        dws = []
        for mi in range(dm):
            w = Wdw[mi]
            acc = fX[0] * w[:, 0:1]
            for k in range(1, K):
                acc = acc + fX[k] * w[:, k:k + 1]
            dws.append(acc)
        dwT = jnp.concatenate(dws, axis=0)
        out = jnp.dot(Wpw_ref[...], dwT, preferred_element_type=jnp.float32) \
            + bpw_ref[...]

        if with_global:
            Wg1 = Wg1_ref[...]
            g = (Wg1[:, 0:1] * repT[0:1, :] + Wg1[:, 1:2] * repT[1:2, :]
                 + Wg1[:, 2:3] * repT[2:3, :]) + bg1_ref[...]
            g = _elu(g)
            g = _elu(jnp.dot(Wg2_ref[...], g,
                             preferred_element_type=jnp.float32)
                     + bg2_ref[...])
            out = jnp.concatenate([g, out], axis=0)

        out_ref[0] = out

    return body


def _xconv_layer(pts, ptsT, ftsT, p, K, D, P, dm, with_global):
    B, N, _ = pts.shape
    Cin = 0 if ftsT is None else ftsT.shape[1]
    Cd = p['Wd1'].shape[1]
    Cmid = Cd + Cin
    Cout = p['Wpw'].shape[1]
    Cg = p['Wg1'].shape[1] if with_global else 0

    P_tile = 512 if P > 384 else P

    catT = ptsT if ftsT is None else jnp.concatenate([ptsT, ftsT], axis=1)
    Wdw_p = jnp.transpose(p['Wdw'], (2, 1, 0))
    WpwT = p['Wpw'].reshape(Cmid, dm, Cout).transpose(1, 0, 2) \
        .reshape(dm * Cmid, Cout).T

    def bcol(b):
        return b.reshape(-1, 1)

    ops = [pts, ptsT, catT,
           p['Wd1'].T, bcol(p['bd1']), p['Wd2'].T, bcol(p['bd2']),
           p['Wx0'].T, bcol(p['bx0']), p['Wx1'].T, bcol(p['bx1']),
           p['Wx2'].T, bcol(p['bx2']),
           Wdw_p, WpwT, bcol(p['bpw'])]
    if with_global:
        ops += [p['Wg1'].T, bcol(p['bg1']), p['Wg2'].T, bcol(p['bg2'])]

    def wspec(arr):
        zeros = (0,) * arr.ndim
        return pl.BlockSpec(arr.shape, lambda b, j, _z=zeros: _z)

    in_specs = [
        pl.BlockSpec((1, N, 3), lambda b, j: (b, 0, 0)),
        pl.BlockSpec((1, 3, P_tile), lambda b, j: (b, 0, j)),
        pl.BlockSpec((1, 3 + Cin, N), lambda b, j: (b, 0, 0)),
    ] + [wspec(a) for a in ops[3:]]

    fn = _make_layer_kernel(K, D, dm, Cin, with_global, N, P_tile)
    return pl.pallas_call(
        fn,
        grid=(B, P // P_tile),
        in_specs=in_specs,
        out_specs=pl.BlockSpec((1, Cg + Cout, P_tile), lambda b, j: (b, 0, j)),
        out_shape=jax.ShapeDtypeStruct((B, Cg + Cout, P), jnp.float32),
        compiler_params=pltpu.CompilerParams(
            dimension_semantics=("parallel", "parallel")),
    )(*ops)


def kernel(pc, params):
    pcT = jnp.transpose(pc, (0, 2, 1))
    N = pc.shape[1]
    ftsT = None
    for (K, D, P, _Cin, _Cout, _Cd, dm, wg), p in zip(_LAYER_CFGS, params):
        P_l = N if P == -1 else P
        ftsT = _xconv_layer(pc[:, :N, :], pcT[:, :, :N], ftsT, p,
                            K, D, P_l, dm, wg)
        N = P_l
    return jnp.transpose(ftsT, (0, 2, 1))


# speedup vs baseline: 17.9992x; 1.0184x over previous
import jax
import jax.numpy as jnp
from jax.experimental import pallas as pl
from jax.experimental.pallas import tpu as pltpu

_LAYER_CFGS = [
    (8, 1, -1, 0, 48, 24, 4, False),
    (12, 2, 384, 48, 96, 12, 2, False),
    (16, 2, 128, 96, 192, 24, 2, False),
    (16, 3, -1, 192, 384, 48, 2, True),
]


def _elu(x):
    return jnp.where(x > 0, x, jnp.exp(x) - 1.0)


def _make_layer_kernel(K, D, dm, Cin, with_global, N, P_tile):
    KD = K * D

    def body(pts_ref, repT_ref, catT_ref, Wd1_ref, bd1_ref, Wd2_ref, bd2_ref,
             Wx0_ref, bx0_ref, Wx1_ref, bx1_ref, Wx2_ref, bx2_ref,
             Wdw_ref, Wpw_ref, bpw_ref, *rest):
        if with_global:
            Wg1_ref, bg1_ref, Wg2_ref, bg2_ref, out_ref = rest
        else:
            (out_ref,) = rest

        pts = pts_ref[0]
        repT = repT_ref[0]
        catT = catT_ref[0]
        Ccat = catT.shape[0]

        def _dist2():
            dx = pts[:, 0:1] - repT[0:1, :]
            dy = pts[:, 1:2] - repT[1:2, :]
            dz = pts[:, 2:3] - repT[2:3, :]
            return (dx * dx + dy * dy) + dz * dz

        iota = jax.lax.broadcasted_iota(jnp.int32, (N, P_tile), 0)
        inf = jnp.float32(jnp.inf)
        ones_cnt = jnp.ones((1, N), jnp.float32)
        d2m = _dist2()
        nn_fast = []
        cntmax = jnp.zeros((1, P_tile), jnp.float32)
        for t in range(KD):
            m = jnp.min(d2m, axis=0, keepdims=True)
            eq = d2m == m
            eqf = eq.astype(jnp.float32)
            if t % D == 0:
                gat = jnp.dot(catT, eqf, preferred_element_type=jnp.float32)
                nn_fast.append(gat)
            cnt = jnp.dot(ones_cnt, eqf, preferred_element_type=jnp.float32)
            cntmax = jnp.maximum(cntmax, cnt)
            d2m = jnp.where(eq, inf, d2m)
        fast = jnp.concatenate(nn_fast, axis=0)
        maxcnt = jnp.max(cntmax)

        def _exact(_):
            d2e = _dist2()
            outs = []
            for t in range(KD):
                m = jnp.min(d2e, axis=0, keepdims=True)
                cand = jnp.where(d2e == m, iota, N)
                sel = jnp.min(cand, axis=0, keepdims=True)
                onehot = iota == sel
                if t % D == 0:
                    outs.append(jnp.dot(catT, onehot.astype(jnp.float32),
                                        preferred_element_type=jnp.float32))
                d2e = jnp.where(onehot, inf, d2e)
            return jnp.concatenate(outs, axis=0)

        gathered = jax.lax.cond(maxcnt > 1.5, _exact, lambda _: fast, None)
        nnT = [gathered[k * Ccat:(k + 1) * Ccat] for k in range(K)]

        locT = [g[0:3, :] - repT for g in nnT]
        lfT = jnp.concatenate(locT, axis=0)
        locT_all = jnp.concatenate(locT, axis=1)

        Wd1 = Wd1_ref[...]
        h = (Wd1[:, 0:1] * locT_all[0:1, :] + Wd1[:, 1:2] * locT_all[1:2, :]
             + Wd1[:, 2:3] * locT_all[2:3, :]) + bd1_ref[...]
        h = _elu(h)
        h = _elu(jnp.dot(Wd2_ref[...], h, preferred_element_type=jnp.float32)
                 + bd2_ref[...])

        X = _elu(jnp.dot(Wx0_ref[...], lfT, preferred_element_type=jnp.float32)
                 + bx0_ref[...])
        X = _elu(jnp.dot(Wx1_ref[...], X, preferred_element_type=jnp.float32)
                 + bx1_ref[...])
        X = jnp.dot(Wx2_ref[...], X, preferred_element_type=jnp.float32) \
            + bx2_ref[...]

        H = []
        for j in range(K):
            hj = h[:, j * P_tile:(j + 1) * P_tile]
            if Cin:
                hj = jnp.concatenate([hj, nnT[j][3:, :]], axis=0)
            H.append(hj)

        fX = []
        for k in range(K):
            acc = X[k * K:k * K + 1, :] * H[0]
            for j in range(1, K):
                acc = acc + X[k * K + j:k * K + j + 1, :] * H[j]
            fX.append(acc)

        Wdw = Wdw_ref[...]
        dws = []
        for mi in range(dm):
            w = Wdw[mi]
            acc = fX[0] * w[:, 0:1]
            for k in range(1, K):
                acc = acc + fX[k] * w[:, k:k + 1]
            dws.append(acc)
        dwT = jnp.concatenate(dws, axis=0)
        out = jnp.dot(Wpw_ref[...], dwT, preferred_element_type=jnp.float32) \
            + bpw_ref[...]

        if with_global:
            Wg1 = Wg1_ref[...]
            g = (Wg1[:, 0:1] * repT[0:1, :] + Wg1[:, 1:2] * repT[1:2, :]
                 + Wg1[:, 2:3] * repT[2:3, :]) + bg1_ref[...]
            g = _elu(g)
            g = _elu(jnp.dot(Wg2_ref[...], g,
                             preferred_element_type=jnp.float32)
                     + bg2_ref[...])
            out = jnp.concatenate([g, out], axis=0)

        out_ref[0] = out

    return body


def _xconv_layer(pts, ptsT, ftsT, p, K, D, P, dm, with_global):
    B, N, _ = pts.shape
    Cin = 0 if ftsT is None else ftsT.shape[1]
    Cd = p['Wd1'].shape[1]
    Cmid = Cd + Cin
    Cout = p['Wpw'].shape[1]
    Cg = p['Wg1'].shape[1] if with_global else 0

    P_tile = 1024 if P > 384 else P

    catT = ptsT if ftsT is None else jnp.concatenate([ptsT, ftsT], axis=1)
    Wdw_p = jnp.transpose(p['Wdw'], (2, 1, 0))
    WpwT = p['Wpw'].reshape(Cmid, dm, Cout).transpose(1, 0, 2) \
        .reshape(dm * Cmid, Cout).T

    def bcol(b):
        return b.reshape(-1, 1)

    ops = [pts, ptsT, catT,
           p['Wd1'].T, bcol(p['bd1']), p['Wd2'].T, bcol(p['bd2']),
           p['Wx0'].T, bcol(p['bx0']), p['Wx1'].T, bcol(p['bx1']),
           p['Wx2'].T, bcol(p['bx2']),
           Wdw_p, WpwT, bcol(p['bpw'])]
    if with_global:
        ops += [p['Wg1'].T, bcol(p['bg1']), p['Wg2'].T, bcol(p['bg2'])]

    def wspec(arr):
        zeros = (0,) * arr.ndim
        return pl.BlockSpec(arr.shape, lambda b, j, _z=zeros: _z)

    in_specs = [
        pl.BlockSpec((1, N, 3), lambda b, j: (b, 0, 0)),
        pl.BlockSpec((1, 3, P_tile), lambda b, j: (b, 0, j)),
        pl.BlockSpec((1, 3 + Cin, N), lambda b, j: (b, 0, 0)),
    ] + [wspec(a) for a in ops[3:]]

    fn = _make_layer_kernel(K, D, dm, Cin, with_global, N, P_tile)
    return pl.pallas_call(
        fn,
        grid=(B, P // P_tile),
        in_specs=in_specs,
        out_specs=pl.BlockSpec((1, Cg + Cout, P_tile), lambda b, j: (b, 0, j)),
        out_shape=jax.ShapeDtypeStruct((B, Cg + Cout, P), jnp.float32),
        compiler_params=pltpu.CompilerParams(
            dimension_semantics=("parallel", "parallel")),
    )(*ops)


def kernel(pc, params):
    pcT = jnp.transpose(pc, (0, 2, 1))
    N = pc.shape[1]
    ftsT = None
    for (K, D, P, _Cin, _Cout, _Cd, dm, wg), p in zip(_LAYER_CFGS, params):
        P_l = N if P == -1 else P
        ftsT = _xconv_layer(pc[:, :N, :], pcT[:, :, :N], ftsT, p,
                            K, D, P_l, dm, wg)
        N = P_l
    return jnp.transpose(ftsT, (0, 2, 1))
